# Initial kernel scaffold; baseline (speedup 1.0000x reference)
import jax
import jax.numpy as jnp
from jax.experimental import pallas as pl


def kernel(x, edge_index, edge_label_index, Wl0, Wr0, att0, b0, g0, be0, Wl1, Wr1, att1, b1, g1, be1):
    raise NotImplementedError("write your pallas kernel here")


# speedup vs baseline: 6.3569x; 6.3569x over previous
import dataclasses
import functools

import jax
import jax.numpy as jnp
from jax import lax
from jax.experimental import pallas as pl
from jax.experimental.pallas import tpu as pltpu
from jax.experimental.pallas import tpu_sc as plsc

N = 10000
D = 128
NEG = 0.2
EPS = 1e-5
ALPHA = 1.0 / 3.0
NC, NS, L = 2, 16, 16
NW = NC * NS
EB = 64
LB = 128
ROWS_PER_SUB = 632
NACC = NS * ROWS_PER_SUB
E_REAL = 320000 + N
NBLK = -(-E_REAL // (NW * EB))
E_PAD = NW * NBLK * EB
NL_REAL = 10000
NLBLK = -(-NL_REAL // (NW * LB))
NL_PAD = NW * NLBLK * LB
NZCHUNK = -(-ROWS_PER_SUB // EB)

_mesh = plsc.VectorSubcoreMesh(core_axis_name="c", subcore_axis_name="s")
_sc_params = pltpu.CompilerParams()
if "needs_layout_passes" in pltpu.CompilerParams.__dataclass_fields__:
    _sc_params = dataclasses.replace(_sc_params, needs_layout_passes=False)


def _edge_pass(xl, xr_pad, att, srcp, dstp):
    @functools.partial(
        pl.kernel,
        mesh=_mesh,
        out_type=[
            jax.ShapeDtypeStruct((NC, NACC, D), jnp.float32),
            jax.ShapeDtypeStruct((NC, NACC, L), jnp.float32),
        ],
        scratch_types=[
            pltpu.VMEM((1, EB), jnp.int32),
            pltpu.VMEM((1, EB), jnp.int32),
            pltpu.VMEM((EB, D), jnp.float32),
            pltpu.VMEM((EB, D), jnp.float32),
            pltpu.VMEM((EB, L), jnp.float32),
            pltpu.VMEM((D,), jnp.float32),
            pltpu.VMEM((1, EB), jnp.int32),
            pltpu.VMEM_SHARED((NACC, D), jnp.float32),
            pltpu.VMEM_SHARED((NACC, L), jnp.float32),
        ],
        compiler_params=_sc_params,
    )
    def k(xl_hbm, xr_hbm, att_hbm, src_hbm, dst_hbm,
          acc_out, den_out, sidx, didx, xlr, xrr, dval, attv, zidx,
          accs, dens):
        cid = lax.axis_index("c")
        sid = lax.axis_index("s")
        wid = cid * NS + sid
        base = sid * ROWS_PER_SUB
        iota = lax.iota(jnp.int32, L)
        zv = jnp.zeros((L,), jnp.float32)

        @pl.loop(0, EB)
        def _zrow(j):
            for c in range(D // L):
                xlr[j, pl.ds(c * L, L)] = zv
            dval[j, :] = zv

        def set_zidx(row0):
            for g in range(EB // L):
                v = jnp.minimum(row0 + (g * L) + iota, NACC - 1)
                zidx[0, pl.ds(g * L, L)] = v

        @pl.loop(0, NZCHUNK)
        def _zfill(t):
            set_zidx(base + t * EB)
            pltpu.sync_copy(xlr, accs.at[zidx.at[0]])
            pltpu.sync_copy(dval, dens.at[zidx.at[0]])

        pltpu.sync_copy(att_hbm, attv)
        att_c = [attv[pl.ds(c * L, L)] for c in range(D // L)]
        plsc.subcore_barrier()

        @pl.loop(0, NBLK)
        def _blk(blk):
            pltpu.sync_copy(src_hbm.at[wid, blk], sidx)
            pltpu.sync_copy(dst_hbm.at[wid, blk], didx)
            pltpu.sync_copy(xl_hbm.at[sidx.at[0]], xlr)
            pltpu.sync_copy(xr_hbm.at[didx.at[0]], xrr)

            @pl.loop(0, EB)
            def _edge(j):
                acc = jnp.zeros((L,), jnp.float32)
                xlc = []
                for c in range(D // L):
                    xc = xlr[j, pl.ds(c * L, L)]
                    xlc.append(xc)
                    v = xc + xrr[j, pl.ds(c * L, L)]
                    m = jnp.maximum(v, NEG * v)
                    acc = acc + m * att_c[c]
                e = jnp.sum(acc)
                wb = jnp.exp(jnp.full((L,), e, jnp.float32))
                for c in range(D // L):
                    xlr[j, pl.ds(c * L, L)] = xlc[c] * wb
                dval[j, :] = wb

            pltpu.sync_copy(xlr, accs.at[didx.at[0]], add=True)
            pltpu.sync_copy(dval, dens.at[didx.at[0]], add=True)

        plsc.subcore_barrier()
        n_full, rem = divmod(ROWS_PER_SUB, EB)
        chunks = [(t * EB, EB) for t in range(n_full)]
        if rem:
            chunks.append((n_full * EB, rem))
        for off, sz in chunks:
            set_zidx(base + off)
            pltpu.sync_copy(accs.at[zidx.at[0]], xlr)
            pltpu.sync_copy(dens.at[zidx.at[0]], dval)
            pltpu.sync_copy(xlr.at[pl.ds(0, sz)],
                            acc_out.at[cid, pl.ds(base + off, sz)])
            pltpu.sync_copy(dval.at[pl.ds(0, sz)],
                            den_out.at[cid, pl.ds(base + off, sz)])

    return k(xl, xr_pad, att, srcp, dstp)


def _project(x, Wl, Wr):
    def body(x_ref, wl_ref, wr_ref, xl_ref, xr_ref):
        xv = x_ref[...]
        xl_ref[...] = jnp.dot(xv, wl_ref[...], preferred_element_type=jnp.float32)
        xr_ref[...] = jnp.dot(xv, wr_ref[...], preferred_element_type=jnp.float32)

    return pl.pallas_call(
        body,
        out_shape=[jax.ShapeDtypeStruct((N, D), jnp.float32)] * 2,
    )(x, Wl, Wr)


def _combine_bn(acc0, acc1, den0, den1, bias, g, be):
    def body(a0, a1, d0, d1, b_ref, g_ref, be_ref, h_ref):
        h = (a0[...] + a1[...]) / (d0[...] + d1[...]) + b_ref[...]
        mu = jnp.mean(h, axis=0)
        hc = h - mu
        var = jnp.mean(hc * hc, axis=0)
        h_ref[...] = g_ref[...] * hc * lax.rsqrt(var + EPS) + be_ref[...]

    return pl.pallas_call(
        body,
        out_shape=jax.ShapeDtypeStruct((N, D), jnp.float32),
    )(acc0, acc1, den0, den1, bias[None, :], g[None, :], be[None, :])


def _mix(h1, h2):
    def body(h1_ref, h2_ref, o_ref):
        o_ref[...] = ALPHA * (h1_ref[...] + h2_ref[...])

    return pl.pallas_call(
        body,
        out_shape=jax.ShapeDtypeStruct((N, D), jnp.float32),
    )(h1, h2)


def _label_dot(out, lsrc, ldst):
    @functools.partial(
        pl.kernel,
        mesh=_mesh,
        out_type=jax.ShapeDtypeStruct((NW, NLBLK, LB, L), jnp.float32),
        scratch_types=[
            pltpu.VMEM((1, LB), jnp.int32),
            pltpu.VMEM((1, LB), jnp.int32),
            pltpu.VMEM((LB, D), jnp.float32),
            pltpu.VMEM((LB, D), jnp.float32),
            pltpu.VMEM((LB, L), jnp.float32),
        ],
        compiler_params=_sc_params,
    )
    def k(tab_hbm, s_hbm, d_hbm, o_hbm, sidx, didx, ur, vr, obuf):
        cid = lax.axis_index("c")
        sid = lax.axis_index("s")
        wid = cid * NS + sid

        @pl.loop(0, NLBLK)
        def _blk(blk):
            pltpu.sync_copy(s_hbm.at[wid, blk], sidx)
            pltpu.sync_copy(d_hbm.at[wid, blk], didx)
            pltpu.sync_copy(tab_hbm.at[sidx.at[0]], ur)
            pltpu.sync_copy(tab_hbm.at[didx.at[0]], vr)

            @pl.loop(0, LB)
            def _dot(j):
                acc = jnp.zeros((L,), jnp.float32)
                for c in range(D // L):
                    acc = acc + ur[j, pl.ds(c * L, L)] * vr[j, pl.ds(c * L, L)]
                obuf[j, :] = acc

            pltpu.sync_copy(obuf, o_hbm.at[wid, blk])

    return k(out, lsrc, ldst)


def _lane_sum(p):
    def body(p_ref, o_ref):
        o_ref[...] = jnp.sum(p_ref[...], axis=-1)

    return pl.pallas_call(
        body,
        out_shape=jax.ShapeDtypeStruct((NL_PAD,), jnp.float32),
    )(p.reshape(NL_PAD, L))


def kernel(x, edge_index, edge_label_index, Wl0, Wr0, att0, b0, g0, be0,
           Wl1, Wr1, att1, b1, g1, be1):
    i32 = edge_index.dtype
    loops = jnp.arange(N, dtype=i32)
    src = jnp.concatenate(
        [edge_index[0], loops, jnp.zeros((E_PAD - E_REAL,), i32)])
    dst = jnp.concatenate(
        [edge_index[1], loops, jnp.full((E_PAD - E_REAL,), N, i32)])
    srcp = src.reshape(NW, NBLK, 1, EB)
    dstp = dst.reshape(NW, NBLK, 1, EB)

    lsrc = jnp.concatenate(
        [edge_label_index[0], jnp.zeros((NL_PAD - NL_REAL,), i32)]
    ).reshape(NW, NLBLK, 1, LB)
    ldst = jnp.concatenate(
        [edge_label_index[1], jnp.zeros((NL_PAD - NL_REAL,), i32)]
    ).reshape(NW, NLBLK, 1, LB)

    def layer(h, Wl, Wr, att, bias, g, be):
        xl, xr = _project(h, Wl, Wr)
        xr_pad = jnp.pad(xr, ((0, NACC - N), (0, 0)))
        acc, den = _edge_pass(xl, xr_pad, att, srcp, dstp)
        return _combine_bn(acc[0, :N], acc[1, :N],
                           den[0, :N, :1], den[1, :N, :1], bias, g, be)

    h1 = layer(x, Wl0, Wr0, att0, b0, g0, be0)
    h2 = layer(h1, Wl1, Wr1, att1, b1, g1, be1)
    out = _mix(h1, h2)
    return _lane_sum(_label_dot(out, lsrc, ldst))[:NL_REAL]
